# Initial kernel scaffold; baseline (speedup 1.0000x reference)
import jax
import jax.numpy as jnp
from jax.experimental import pallas as pl


def kernel(weight, input_ids, labels):
    raise NotImplementedError("write your pallas kernel here")


# speedup vs baseline: 1.2101x; 1.2101x over previous
import functools
import math

import jax
import jax.numpy as jnp
from jax import lax
from jax.experimental import pallas as pl
from jax.experimental.pallas import tpu as pltpu
from jax.experimental.pallas import tpu_sc as plsc

D = 768
HEAD_DIM = 64
HALF = HEAD_DIM // 2
THETA = 10000.0
MASK_VAL = -2.3819763e+38
SCALE = float(D) ** 0.5

NW = 32
CHUNK = 128
LANES = 16


def _gather_body(n_ids, table_hbm, ids_hbm, out_hbm, idx_v, rows_v, sem):
    b_per_w = n_ids // NW
    n_chunks = b_per_w // CHUNK
    wid = lax.axis_index("s") * 2 + lax.axis_index("c")
    base = wid * b_per_w

    def chunk_body(c, carry):
        off = pl.multiple_of(base + c * CHUNK, CHUNK)
        pltpu.sync_copy(ids_hbm.at[pl.ds(off, CHUNK)], idx_v)
        pltpu.async_copy(table_hbm.at[idx_v], rows_v, sem).wait()

        def row_body(r, carry2):
            for j in range(D // LANES):
                sl = pl.ds(j * LANES, LANES)
                rows_v[r, sl] = rows_v[r, sl] * SCALE
            return carry2

        lax.fori_loop(0, CHUNK, row_body, 0)
        pltpu.sync_copy(rows_v, out_hbm.at[pl.ds(off, CHUNK)])
        return carry

    lax.fori_loop(0, n_chunks, chunk_body, 0)


def _sc_gather(weight, ids_flat):
    n_ids = ids_flat.shape[0]
    mesh = plsc.VectorSubcoreMesh(core_axis_name="c", subcore_axis_name="s")
    k = functools.partial(
        pl.kernel,
        mesh=mesh,
        out_type=jax.ShapeDtypeStruct((n_ids, D), jnp.float32),
        scratch_types=[
            pltpu.VMEM((CHUNK,), jnp.int32),
            pltpu.VMEM((CHUNK, D), jnp.float32),
            pltpu.SemaphoreType.DMA,
        ],
    )(functools.partial(_gather_body, n_ids))
    return k(weight, ids_flat)


def _maskfreq_body(seq_len, mrows, mask_ref, cos_ref, sin_ref):
    i = pl.program_id(0)
    r0 = i * mrows
    row = lax.broadcasted_iota(jnp.int32, (mrows, seq_len), 0) + r0
    col = lax.broadcasted_iota(jnp.int32, (mrows, seq_len), 1)
    mask_ref[0, 0] = jnp.where(col > row, jnp.float32(MASK_VAL),
                               jnp.float32(0.0))
    t = (lax.broadcasted_iota(jnp.int32, (mrows, HALF), 0) + r0
         ).astype(jnp.float32)
    kk = lax.broadcasted_iota(jnp.int32, (mrows, HALF), 1).astype(jnp.float32)
    log_theta = jnp.float32(math.log(THETA))
    ang = t * jnp.exp(kk * (jnp.float32(1.0 / HALF)) * (-log_theta))
    cos_ref[...] = jnp.cos(ang)
    sin_ref[...] = jnp.sin(ang)


def _tc_mask_freqs(seq_len):
    mrows = 256
    body = functools.partial(_maskfreq_body, seq_len, mrows)
    return pl.pallas_call(
        body,
        grid=(seq_len // mrows,),
        out_shape=(
            jax.ShapeDtypeStruct((1, 1, seq_len, seq_len), jnp.float32),
            jax.ShapeDtypeStruct((seq_len, HALF), jnp.float32),
            jax.ShapeDtypeStruct((seq_len, HALF), jnp.float32),
        ),
        out_specs=(
            pl.BlockSpec((1, 1, mrows, seq_len), lambda i: (0, 0, i, 0)),
            pl.BlockSpec((mrows, HALF), lambda i: (i, 0)),
            pl.BlockSpec((mrows, HALF), lambda i: (i, 0)),
        ),
    )()


def kernel(weight, input_ids, labels):
    b, s = input_ids.shape
    ids_flat = input_ids.reshape(-1).astype(jnp.int32)
    rows = _sc_gather(weight, ids_flat)
    hidden = rows.reshape(b, s, D)
    mask, cos, sin = _tc_mask_freqs(s)
    freqs_cis = jax.lax.complex(cos, sin)
    return (hidden, freqs_cis, mask, labels)


# speedup vs baseline: 1.3023x; 1.0762x over previous
import functools
import math

import jax
import jax.numpy as jnp
from jax import lax
from jax.experimental import pallas as pl
from jax.experimental.pallas import tpu as pltpu
from jax.experimental.pallas import tpu_sc as plsc

D = 768
HEAD_DIM = 64
HALF = HEAD_DIM // 2
THETA = 10000.0
MASK_VAL = -2.3819763e+38
SCALE = float(D) ** 0.5

NW = 32
CHUNK = 32
NBUF = 4
LANES = 16


def _scale_rows(rows_ref):
    def row_body(r, carry):
        for j in range(D // LANES):
            sl = pl.ds(j * LANES, LANES)
            rows_ref[r, sl] = rows_ref[r, sl] * SCALE
        return carry

    lax.fori_loop(0, CHUNK, row_body, 0)


def _gather_body(n_ids, table_hbm, ids_hbm, out_hbm, idx_all, *bufs_sems):
    rows = bufs_sems[:NBUF]
    gsems = bufs_sems[NBUF:2 * NBUF]
    ssems = bufs_sems[2 * NBUF:3 * NBUF]
    b_per_w = n_ids // NW
    n_chunks = b_per_w // CHUNK
    wid = lax.axis_index("s") * 2 + lax.axis_index("c")
    base = wid * b_per_w

    pltpu.sync_copy(ids_hbm.at[wid], idx_all)

    hg = {}
    hs = {}

    def issue_gather(c):
        b = c % NBUF
        hg[b] = pltpu.async_copy(table_hbm.at[idx_all.at[c]], rows[b],
                                 gsems[b])

    for c in range(min(NBUF, n_chunks)):
        issue_gather(c)
    for c in range(n_chunks):
        b = c % NBUF
        hg[b].wait()
        _scale_rows(rows[b])
        off = pl.multiple_of(base + c * CHUNK, CHUNK)
        hs[c] = pltpu.async_copy(rows[b], out_hbm.at[pl.ds(off, CHUNK)],
                                 ssems[b])
        p = c - 2
        if p >= 0 and p + NBUF < n_chunks:
            hs.pop(p).wait()
            issue_gather(p + NBUF)
    for c in sorted(hs):
        hs.pop(c).wait()


def _sc_gather(weight, ids_flat):
    n_ids = ids_flat.shape[0]
    b_per_w = n_ids // NW
    ids3 = ids_flat.reshape(NW, b_per_w // CHUNK, CHUNK)
    mesh = plsc.VectorSubcoreMesh(core_axis_name="c", subcore_axis_name="s")
    scratch = [pltpu.VMEM((b_per_w // CHUNK, CHUNK), jnp.int32)]
    scratch += [pltpu.VMEM((CHUNK, D), jnp.float32) for _ in range(NBUF)]
    scratch += [pltpu.SemaphoreType.DMA for _ in range(2 * NBUF)]
    k = functools.partial(
        pl.kernel,
        mesh=mesh,
        out_type=jax.ShapeDtypeStruct((n_ids, D), jnp.float32),
        scratch_types=scratch,
    )(functools.partial(_gather_body, n_ids))
    return k(weight, ids3)


def _maskfreq_body(seq_len, mrows, mask_ref, cos_ref, sin_ref):
    i = pl.program_id(0)
    r0 = i * mrows
    row = lax.broadcasted_iota(jnp.int32, (mrows, seq_len), 0) + r0
    col = lax.broadcasted_iota(jnp.int32, (mrows, seq_len), 1)
    mask_ref[0, 0] = jnp.where(col > row, jnp.float32(MASK_VAL),
                               jnp.float32(0.0))
    t = (lax.broadcasted_iota(jnp.int32, (mrows, HALF), 0) + r0
         ).astype(jnp.float32)
    kk = lax.broadcasted_iota(jnp.int32, (mrows, HALF), 1).astype(jnp.float32)
    log_theta = jnp.float32(math.log(THETA))
    ang = t * jnp.exp(kk * (jnp.float32(1.0 / HALF)) * (-log_theta))
    cos_ref[...] = jnp.cos(ang)
    sin_ref[...] = jnp.sin(ang)


def _tc_mask_freqs(seq_len):
    mrows = 256
    body = functools.partial(_maskfreq_body, seq_len, mrows)
    return pl.pallas_call(
        body,
        grid=(seq_len // mrows,),
        out_shape=(
            jax.ShapeDtypeStruct((1, 1, seq_len, seq_len), jnp.float32),
            jax.ShapeDtypeStruct((seq_len, HALF), jnp.float32),
            jax.ShapeDtypeStruct((seq_len, HALF), jnp.float32),
        ),
        out_specs=(
            pl.BlockSpec((1, 1, mrows, seq_len), lambda i: (0, 0, i, 0)),
            pl.BlockSpec((mrows, HALF), lambda i: (i, 0)),
            pl.BlockSpec((mrows, HALF), lambda i: (i, 0)),
        ),
    )()


def kernel(weight, input_ids, labels):
    b, s = input_ids.shape
    ids_flat = input_ids.reshape(-1).astype(jnp.int32)
    rows = _sc_gather(weight, ids_flat)
    hidden = rows.reshape(b, s, D)
    mask, cos, sin = _tc_mask_freqs(s)
    freqs_cis = jax.lax.complex(cos, sin)
    return (hidden, freqs_cis, mask, labels)


# speedup vs baseline: 1.3904x; 1.0677x over previous
import functools
import math

import jax
import jax.numpy as jnp
from jax import lax
from jax.experimental import pallas as pl
from jax.experimental.pallas import tpu as pltpu
from jax.experimental.pallas import tpu_sc as plsc

D = 768
HEAD_DIM = 64
HALF = HEAD_DIM // 2
THETA = 10000.0
MASK_VAL = -2.3819763e+38
SCALE = float(D) ** 0.5

NW = 32
CHUNK = 32
NBUF = 5
LANES = 16


def _scale_rows(rows_ref):
    def row_body(r, carry):
        for j in range(D // LANES):
            sl = pl.ds(j * LANES, LANES)
            rows_ref[r, sl] = rows_ref[r, sl] * SCALE
        return carry

    lax.fori_loop(0, CHUNK, row_body, 0)


def _gather_body(bsz, seq, table_hbm, ids_hbm, out_hbm, idx_all, *bufs_sems):
    rows = bufs_sems[:NBUF]
    gsems = bufs_sems[NBUF:2 * NBUF]
    ssems = bufs_sems[2 * NBUF:3 * NBUF]
    b_per_w = (bsz * seq) // NW
    n_chunks = b_per_w // CHUNK
    wid = lax.axis_index("s") * 2 + lax.axis_index("c")
    base = wid * b_per_w
    bidx = base // seq
    loff0 = base % seq

    pltpu.sync_copy(ids_hbm.at[wid], idx_all)

    hg = {}
    hs = {}

    def issue_gather(c):
        b = c % NBUF
        hg[c] = pltpu.async_copy(table_hbm.at[idx_all.at[c]], rows[b],
                                 gsems[b])

    def issue_store(c):
        b = c % NBUF
        loff = pl.multiple_of(loff0 + c * CHUNK, CHUNK)
        hs[c] = pltpu.async_copy(rows[b], out_hbm.at[bidx, pl.ds(loff, CHUNK)],
                                 ssems[b])

    for c in range(min(NBUF - 2, n_chunks)):
        issue_gather(c)
    for c in range(n_chunks):
        hg.pop(c).wait()
        nxt = c + NBUF - 2
        if nxt < n_chunks:
            if nxt - NBUF >= 0:
                hs.pop(nxt - NBUF).wait()
            issue_gather(nxt)
        _scale_rows(rows[c % NBUF])
        issue_store(c)
    for c in sorted(hs):
        hs.pop(c).wait()


def _sc_gather(weight, input_ids):
    bsz, seq = input_ids.shape
    n_ids = bsz * seq
    b_per_w = n_ids // NW
    ids3 = input_ids.astype(jnp.int32).reshape(NW, b_per_w // CHUNK, CHUNK)
    mesh = plsc.VectorSubcoreMesh(core_axis_name="c", subcore_axis_name="s")
    scratch = [pltpu.VMEM((b_per_w // CHUNK, CHUNK), jnp.int32)]
    scratch += [pltpu.VMEM((CHUNK, D), jnp.float32) for _ in range(NBUF)]
    scratch += [pltpu.SemaphoreType.DMA for _ in range(2 * NBUF)]
    k = functools.partial(
        pl.kernel,
        mesh=mesh,
        out_type=jax.ShapeDtypeStruct((bsz, seq, D), jnp.float32),
        scratch_types=scratch,
    )(functools.partial(_gather_body, bsz, seq))
    return k(weight, ids3)


def _maskfreq_body(seq_len, mrows, lab_in, mask_ref, cos_ref, sin_ref,
                   lab_out):
    i = pl.program_id(0)
    lab_out[...] = lab_in[...]
    r0 = i * mrows
    row = lax.broadcasted_iota(jnp.int32, (mrows, seq_len), 0) + r0
    col = lax.broadcasted_iota(jnp.int32, (mrows, seq_len), 1)
    mask_ref[0, 0] = jnp.where(col > row, jnp.float32(MASK_VAL),
                               jnp.float32(0.0))
    t = (lax.broadcasted_iota(jnp.int32, (mrows, HALF), 0) + r0
         ).astype(jnp.float32)
    kk = lax.broadcasted_iota(jnp.int32, (mrows, HALF), 1).astype(jnp.float32)
    log_theta = jnp.float32(math.log(THETA))
    ang = t * jnp.exp(kk * (jnp.float32(1.0 / HALF)) * (-log_theta))
    cos_ref[...] = jnp.cos(ang)
    sin_ref[...] = jnp.sin(ang)


def _tc_mask_freqs(seq_len, labels):
    mrows = 256
    bsz = labels.shape[0]
    body = functools.partial(_maskfreq_body, seq_len, mrows)
    return pl.pallas_call(
        body,
        grid=(seq_len // mrows,),
        in_specs=(pl.BlockSpec((bsz, seq_len), lambda i: (0, 0)),),
        out_shape=(
            jax.ShapeDtypeStruct((1, 1, seq_len, seq_len), jnp.float32),
            jax.ShapeDtypeStruct((seq_len, HALF), jnp.float32),
            jax.ShapeDtypeStruct((seq_len, HALF), jnp.float32),
            jax.ShapeDtypeStruct((bsz, seq_len), labels.dtype),
        ),
        out_specs=(
            pl.BlockSpec((1, 1, mrows, seq_len), lambda i: (0, 0, i, 0)),
            pl.BlockSpec((mrows, HALF), lambda i: (i, 0)),
            pl.BlockSpec((mrows, HALF), lambda i: (i, 0)),
            pl.BlockSpec((bsz, seq_len), lambda i: (0, 0)),
        ),
    )(labels)


def kernel(weight, input_ids, labels):
    b, s = input_ids.shape
    hidden = _sc_gather(weight, input_ids)
    mask, cos, sin, lab_out = _tc_mask_freqs(s, labels)
    freqs_cis = jax.lax.complex(cos, sin)
    return (hidden, freqs_cis, mask, lab_out)


# speedup vs baseline: 1.4209x; 1.0219x over previous
import functools
import math

import jax
import jax.numpy as jnp
from jax import lax
from jax.experimental import pallas as pl
from jax.experimental.pallas import tpu as pltpu
from jax.experimental.pallas import tpu_sc as plsc

D = 768
HEAD_DIM = 64
HALF = HEAD_DIM // 2
THETA = 10000.0
MASK_VAL = -2.3819763e+38
SCALE = float(D) ** 0.5

NW = 32
CHUNK = 32
NBUF = 5
LEAD = NBUF - 2
LANES = 16


UNROLL = 16


def _scale_rows(rows_ref):
    def row_body(r, carry):
        @plsc.parallel_loop(0, D // LANES, unroll=UNROLL)
        def _(j):
            sl = pl.ds(j * LANES, LANES)
            rows_ref[r, sl] = rows_ref[r, sl] * SCALE

        return carry

    lax.fori_loop(0, CHUNK, row_body, 0)


def _gather_body(bsz, seq, table_hbm, ids_hbm, out_hbm, idx_all, *bufs_sems):
    rows = bufs_sems[:NBUF]
    gsems = bufs_sems[NBUF:2 * NBUF]
    ssems = bufs_sems[2 * NBUF:3 * NBUF]
    b_per_w = (bsz * seq) // NW
    n_chunks = b_per_w // CHUNK
    wid = lax.axis_index("s") * 2 + lax.axis_index("c")
    base = wid * b_per_w
    bidx = base // seq
    loff0 = base % seq

    pltpu.sync_copy(ids_hbm.at[bidx, pl.ds(loff0, b_per_w)], idx_all)

    hg = {}
    hs = {}

    def issue_gather(c):
        b = c % NBUF
        idx = idx_all.at[pl.ds(c * CHUNK, CHUNK)]
        hg[c] = pltpu.async_copy(table_hbm.at[idx], rows[b], gsems[b])

    def issue_store(c):
        b = c % NBUF
        loff = pl.multiple_of(loff0 + c * CHUNK, CHUNK)
        hs[c] = pltpu.async_copy(rows[b], out_hbm.at[bidx, pl.ds(loff, CHUNK)],
                                 ssems[b])

    for c in range(min(LEAD, n_chunks)):
        issue_gather(c)
    for c in range(n_chunks):
        hg.pop(c).wait()
        nxt = c + LEAD
        if nxt < n_chunks:
            if nxt - NBUF >= 0:
                hs.pop(nxt - NBUF).wait()
            issue_gather(nxt)
        _scale_rows(rows[c % NBUF])
        issue_store(c)
    for c in sorted(hs):
        hs.pop(c).wait()


def _sc_gather(weight, input_ids):
    bsz, seq = input_ids.shape
    n_ids = bsz * seq
    b_per_w = n_ids // NW
    mesh = plsc.VectorSubcoreMesh(core_axis_name="c", subcore_axis_name="s")
    scratch = [pltpu.VMEM((b_per_w,), jnp.int32)]
    scratch += [pltpu.VMEM((CHUNK, D), jnp.float32) for _ in range(NBUF)]
    scratch += [pltpu.SemaphoreType.DMA for _ in range(2 * NBUF)]
    k = functools.partial(
        pl.kernel,
        mesh=mesh,
        out_type=jax.ShapeDtypeStruct((bsz, seq, D), jnp.float32),
        scratch_types=scratch,
    )(functools.partial(_gather_body, bsz, seq))
    return k(weight, input_ids.astype(jnp.int32))


def _maskfreq_body(seq_len, mrows, lab_in, mask_ref, cos_ref, sin_ref,
                   lab_out):
    i = pl.program_id(0)
    lab_out[...] = lab_in[...]
    r0 = i * mrows
    row = lax.broadcasted_iota(jnp.int32, (mrows, seq_len), 0) + r0
    col = lax.broadcasted_iota(jnp.int32, (mrows, seq_len), 1)
    mask_ref[0, 0] = jnp.where(col > row, jnp.float32(MASK_VAL),
                               jnp.float32(0.0))
    t = (lax.broadcasted_iota(jnp.int32, (mrows, HALF), 0) + r0
         ).astype(jnp.float32)
    kk = lax.broadcasted_iota(jnp.int32, (mrows, HALF), 1).astype(jnp.float32)
    log_theta = jnp.float32(math.log(THETA))
    ang = t * jnp.exp(kk * (jnp.float32(1.0 / HALF)) * (-log_theta))
    cos_ref[...] = jnp.cos(ang)
    sin_ref[...] = jnp.sin(ang)


def _tc_mask_freqs(seq_len, labels):
    mrows = 256
    bsz = labels.shape[0]
    body = functools.partial(_maskfreq_body, seq_len, mrows)
    return pl.pallas_call(
        body,
        grid=(seq_len // mrows,),
        in_specs=(pl.BlockSpec((bsz, seq_len), lambda i: (0, 0)),),
        out_shape=(
            jax.ShapeDtypeStruct((1, 1, seq_len, seq_len), jnp.float32),
            jax.ShapeDtypeStruct((seq_len, HALF), jnp.float32),
            jax.ShapeDtypeStruct((seq_len, HALF), jnp.float32),
            jax.ShapeDtypeStruct((bsz, seq_len), labels.dtype),
        ),
        out_specs=(
            pl.BlockSpec((1, 1, mrows, seq_len), lambda i: (0, 0, i, 0)),
            pl.BlockSpec((mrows, HALF), lambda i: (i, 0)),
            pl.BlockSpec((mrows, HALF), lambda i: (i, 0)),
            pl.BlockSpec((bsz, seq_len), lambda i: (0, 0)),
        ),
    )(labels)


def kernel(weight, input_ids, labels):
    b, s = input_ids.shape
    hidden = _sc_gather(weight, input_ids)
    mask, cos, sin, lab_out = _tc_mask_freqs(s, labels)
    freqs_cis = jax.lax.complex(cos, sin)
    return (hidden, freqs_cis, mask, lab_out)
